# Initial kernel scaffold; baseline (speedup 1.0000x reference)
import jax
import jax.numpy as jnp
from jax.experimental import pallas as pl


def kernel(x, W):
    raise NotImplementedError("write your pallas kernel here")


# speedup vs baseline: 1.9675x; 1.9675x over previous
import jax
import jax.numpy as jnp
from jax.experimental import pallas as pl
from jax.experimental.pallas import tpu as pltpu

N_EMBD = 128
THRESHOLD = 0.5
EPS = 1e-8


def _body(x_ref, w_ref, p_ref, bt_ref):
    x = x_ref[0]
    w = w_ref[...]
    kq = jax.lax.dot_general(
        x, w, (((1,), (0,)), ((), ())),
        preferred_element_type=jnp.float32,
    )
    k = kq[:, :N_EMBD]
    q = kq[:, N_EMBD:]
    k_prev = pltpu.roll(k, 1, 0)
    qn = jnp.sqrt(jnp.sum(q * q, axis=1, keepdims=True)) + EPS
    kn = jnp.sqrt(jnp.sum(k_prev * k_prev, axis=1, keepdims=True)) + EPS
    cos = jnp.sum((q / qn) * (k_prev / kn), axis=1, keepdims=True)
    p_row = (0.5 * (1.0 - cos)).T
    t_idx = jax.lax.broadcasted_iota(jnp.int32, p_row.shape, 1)
    p_row = jnp.where(t_idx == 0, 1.0, p_row)
    p_ref[0] = p_row
    bt_ref[0] = (p_row >= THRESHOLD).astype(jnp.float32)


def kernel(x, W):
    Bn, T, C = x.shape
    p3, bt3 = pl.pallas_call(
        _body,
        grid=(Bn,),
        in_specs=[
            pl.BlockSpec((1, T, C), lambda i: (i, 0, 0)),
            pl.BlockSpec((C, 2 * C), lambda i: (0, 0)),
        ],
        out_specs=[
            pl.BlockSpec((1, 1, T), lambda i: (i, 0, 0)),
            pl.BlockSpec((1, 1, T), lambda i: (i, 0, 0)),
        ],
        out_shape=[
            jax.ShapeDtypeStruct((Bn, 1, T), jnp.float32),
            jax.ShapeDtypeStruct((Bn, 1, T), jnp.float32),
        ],
        compiler_params=pltpu.CompilerParams(
            dimension_semantics=("arbitrary",),
        ),
    )(x, W)
    return p3.reshape(Bn, T), bt3.reshape(Bn, T)


# speedup vs baseline: 3.8982x; 1.9814x over previous
import jax
import jax.numpy as jnp
from jax.experimental import pallas as pl
from jax.experimental.pallas import tpu as pltpu

N_EMBD = 128
THRESHOLD = 0.5
EPS = 1e-8


def _body(x_ref, w_ref, p_ref, bt_ref):
    x = x_ref[0]
    w = w_ref[...]
    xT = x.T
    kqT = jax.lax.dot_general(
        w, xT, (((0,), (0,)), ((), ())),
        preferred_element_type=jnp.float32,
    )
    kT = kqT[:N_EMBD]
    qT = kqT[N_EMBD:]
    kprevT = pltpu.roll(kT, 1, 1)
    num = jnp.sum(qT * kprevT, axis=0, keepdims=True)
    qq = jnp.sum(qT * qT, axis=0, keepdims=True)
    kk = jnp.sum(kprevT * kprevT, axis=0, keepdims=True)
    cos = num / ((jnp.sqrt(qq) + EPS) * (jnp.sqrt(kk) + EPS))
    p_row = 0.5 * (1.0 - cos)
    t_idx = jax.lax.broadcasted_iota(jnp.int32, p_row.shape, 1)
    p_row = jnp.where(t_idx == 0, 1.0, p_row)
    p_ref[0] = p_row
    bt_ref[0] = (p_row >= THRESHOLD).astype(jnp.float32)


def kernel(x, W):
    Bn, T, C = x.shape
    p3, bt3 = pl.pallas_call(
        _body,
        grid=(Bn,),
        in_specs=[
            pl.BlockSpec((1, T, C), lambda i: (i, 0, 0)),
            pl.BlockSpec((C, 2 * C), lambda i: (0, 0)),
        ],
        out_specs=[
            pl.BlockSpec((1, 1, T), lambda i: (i, 0, 0)),
            pl.BlockSpec((1, 1, T), lambda i: (i, 0, 0)),
        ],
        out_shape=[
            jax.ShapeDtypeStruct((Bn, 1, T), jnp.float32),
            jax.ShapeDtypeStruct((Bn, 1, T), jnp.float32),
        ],
        compiler_params=pltpu.CompilerParams(
            dimension_semantics=("arbitrary",),
        ),
    )(x, W)
    return p3.reshape(Bn, T), bt3.reshape(Bn, T)


# speedup vs baseline: 3.9167x; 1.0047x over previous
import jax
import jax.numpy as jnp
from jax.experimental import pallas as pl
from jax.experimental.pallas import tpu as pltpu

N_EMBD = 128
THRESHOLD = 0.5
EPS = 1e-8


def _body(x_ref, w_ref, p_ref, bt_ref):
    x = x_ref[0]
    w = w_ref[...]
    xT = x.T
    kqT = jax.lax.dot_general(
        w, xT, (((0,), (0,)), ((), ())),
        preferred_element_type=jnp.float32,
    )
    kT = kqT[:N_EMBD]
    qT = kqT[N_EMBD:]
    kprevT = pltpu.roll(kT, 1, 1)
    num = jnp.sum(qT * kprevT, axis=0, keepdims=True)
    qq = jnp.sum(qT * qT, axis=0, keepdims=True)
    kk = jnp.sum(kprevT * kprevT, axis=0, keepdims=True)
    cos = num / ((jnp.sqrt(qq) + EPS) * (jnp.sqrt(kk) + EPS))
    p_row = 0.5 * (1.0 - cos)
    t_idx = jax.lax.broadcasted_iota(jnp.int32, p_row.shape, 1)
    p_row = jnp.where(t_idx == 0, 1.0, p_row)
    p_ref[0] = p_row
    bt_ref[0] = (p_row >= THRESHOLD).astype(jnp.float32)


def kernel(x, W):
    Bn, T, C = x.shape
    p3, bt3 = pl.pallas_call(
        _body,
        grid=(Bn,),
        in_specs=[
            pl.BlockSpec((1, T, C), lambda i: (i, 0, 0)),
            pl.BlockSpec((C, 2 * C), lambda i: (0, 0)),
        ],
        out_specs=[
            pl.BlockSpec((1, 1, T), lambda i: (i, 0, 0)),
            pl.BlockSpec((1, 1, T), lambda i: (i, 0, 0)),
        ],
        out_shape=[
            jax.ShapeDtypeStruct((Bn, 1, T), jnp.float32),
            jax.ShapeDtypeStruct((Bn, 1, T), jnp.float32),
        ],
        compiler_params=pltpu.CompilerParams(
            dimension_semantics=("parallel",),
        ),
    )(x, W)
    return p3.reshape(Bn, T), bt3.reshape(Bn, T)


# speedup vs baseline: 4.6747x; 1.1935x over previous
import jax
import jax.numpy as jnp
from jax.experimental import pallas as pl
from jax.experimental.pallas import tpu as pltpu

N_EMBD = 128
THRESHOLD = 0.5
EPS = 1e-8
ROWS_PER_STEP = 2


def _body(x_ref, w_ref, p_ref, bt_ref):
    R, T, C = x_ref.shape
    x = x_ref[...].reshape(R * T, C)
    w = w_ref[...]
    xT = x.T
    kqT = jax.lax.dot_general(
        w, xT, (((0,), (0,)), ((), ())),
        preferred_element_type=jnp.float32,
    )
    kT = kqT[:N_EMBD]
    qT = kqT[N_EMBD:]
    kprevT = pltpu.roll(kT, 1, 1)
    num = jnp.sum(qT * kprevT, axis=0, keepdims=True)
    qq = jnp.sum(qT * qT, axis=0, keepdims=True)
    kk = jnp.sum(kprevT * kprevT, axis=0, keepdims=True)
    cos = num / ((jnp.sqrt(qq) + EPS) * (jnp.sqrt(kk) + EPS))
    p_row = 0.5 * (1.0 - cos)
    t_idx = jax.lax.broadcasted_iota(jnp.int32, p_row.shape, 1)
    p_row = jnp.where(t_idx % T == 0, 1.0, p_row)
    bt_row = (p_row >= THRESHOLD).astype(jnp.float32)
    for r in range(R):
        p_ref[r : r + 1, 0] = p_row[:, r * T : (r + 1) * T]
        bt_ref[r : r + 1, 0] = bt_row[:, r * T : (r + 1) * T]


def kernel(x, W):
    Bn, T, C = x.shape
    R = ROWS_PER_STEP
    p3, bt3 = pl.pallas_call(
        _body,
        grid=(Bn // R,),
        in_specs=[
            pl.BlockSpec((R, T, C), lambda i: (i, 0, 0)),
            pl.BlockSpec((C, 2 * C), lambda i: (0, 0)),
        ],
        out_specs=[
            pl.BlockSpec((R, 1, T), lambda i: (i, 0, 0)),
            pl.BlockSpec((R, 1, T), lambda i: (i, 0, 0)),
        ],
        out_shape=[
            jax.ShapeDtypeStruct((Bn, 1, T), jnp.float32),
            jax.ShapeDtypeStruct((Bn, 1, T), jnp.float32),
        ],
        compiler_params=pltpu.CompilerParams(
            dimension_semantics=("arbitrary",),
        ),
    )(x, W)
    return p3.reshape(Bn, T), bt3.reshape(Bn, T)


# speedup vs baseline: 4.7857x; 1.0238x over previous
import jax
import jax.numpy as jnp
from jax.experimental import pallas as pl
from jax.experimental.pallas import tpu as pltpu

N_EMBD = 128
THRESHOLD = 0.5
EPS = 1e-8
ROWS_PER_STEP = 4


def _body(x_ref, w_ref, p_ref, bt_ref):
    R, T, C = x_ref.shape
    x = x_ref[...].reshape(R * T, C)
    w = w_ref[...]
    xT = x.T
    kqT = jax.lax.dot_general(
        w, xT, (((0,), (0,)), ((), ())),
        preferred_element_type=jnp.float32,
    )
    kT = kqT[:N_EMBD]
    qT = kqT[N_EMBD:]
    kprevT = pltpu.roll(kT, 1, 1)
    num = jnp.sum(qT * kprevT, axis=0, keepdims=True)
    qq = jnp.sum(qT * qT, axis=0, keepdims=True)
    kk = jnp.sum(kprevT * kprevT, axis=0, keepdims=True)
    cos = num / ((jnp.sqrt(qq) + EPS) * (jnp.sqrt(kk) + EPS))
    p_row = 0.5 * (1.0 - cos)
    t_idx = jax.lax.broadcasted_iota(jnp.int32, p_row.shape, 1)
    p_row = jnp.where(t_idx % T == 0, 1.0, p_row)
    bt_row = (p_row >= THRESHOLD).astype(jnp.float32)
    for r in range(R):
        p_ref[r : r + 1, 0] = p_row[:, r * T : (r + 1) * T]
        bt_ref[r : r + 1, 0] = bt_row[:, r * T : (r + 1) * T]


def kernel(x, W):
    Bn, T, C = x.shape
    R = ROWS_PER_STEP
    p3, bt3 = pl.pallas_call(
        _body,
        grid=(Bn // R,),
        in_specs=[
            pl.BlockSpec((R, T, C), lambda i: (i, 0, 0)),
            pl.BlockSpec((C, 2 * C), lambda i: (0, 0)),
        ],
        out_specs=[
            pl.BlockSpec((R, 1, T), lambda i: (i, 0, 0)),
            pl.BlockSpec((R, 1, T), lambda i: (i, 0, 0)),
        ],
        out_shape=[
            jax.ShapeDtypeStruct((Bn, 1, T), jnp.float32),
            jax.ShapeDtypeStruct((Bn, 1, T), jnp.float32),
        ],
        compiler_params=pltpu.CompilerParams(
            dimension_semantics=("arbitrary",),
        ),
    )(x, W)
    return p3.reshape(Bn, T), bt3.reshape(Bn, T)
